# baseline (speedup 1.0000x reference)
import functools

import jax
import jax.numpy as jnp
from jax import lax
from jax.experimental import pallas as pl
from jax.experimental.pallas import tpu as pltpu
from jax.experimental.pallas import tpu_sc as plsc

_B = 16384
_DIM = 64
_K = 8192

_BB = 128

_NW = 32
_BPW = _B // _NW
_CH = 128
_NCH = _BPW // _CH


def _argmin_body(x_ref, xb16_ref, embed_ref, eb16_ref, ind_ref, dsum_ref,
                 acc_ref):
    i = pl.program_id(0)
    xb = x_ref[...]
    e = embed_ref[...]
    s = lax.dot_general(xb16_ref[...], eb16_ref[...],
                        (((1,), (0,)), ((), ())),
                        preferred_element_type=jnp.float32)
    x2 = jnp.sum(xb * xb, axis=1, keepdims=True)
    e2 = jnp.sum(e * e, axis=0, keepdims=True)
    dist = (x2 - 2.0 * s) + e2
    m = jnp.min(dist, axis=1, keepdims=True)
    cols = lax.broadcasted_iota(jnp.int32, (_BB, _K), 1)
    masked = jnp.where(dist == m, cols, _K)
    ind_ref[...] = jnp.min(masked, axis=1, keepdims=True)

    @pl.when(i == 0)
    def _():
        acc_ref[0, 0] = 0.0

    acc_ref[0, 0] += jnp.sum(m)

    @pl.when(i == pl.num_programs(0) - 1)
    def _():
        dsum_ref[0, 0] = acc_ref[0, 0] / (_B * _DIM)


def _tc_argmin(x, embed):
    return pl.pallas_call(
        _argmin_body,
        grid=(_B // _BB,),
        in_specs=[
            pl.BlockSpec((_BB, _DIM), lambda i: (i, 0)),
            pl.BlockSpec((_BB, _DIM), lambda i: (i, 0)),
            pl.BlockSpec((_DIM, _K), lambda i: (0, 0)),
            pl.BlockSpec((_DIM, _K), lambda i: (0, 0)),
        ],
        out_specs=[
            pl.BlockSpec((_BB, 1), lambda i: (i, 0)),
            pl.BlockSpec(memory_space=pltpu.SMEM),
        ],
        out_shape=[
            jax.ShapeDtypeStruct((_B, 1), jnp.int32),
            jax.ShapeDtypeStruct((1, 1), jnp.float32),
        ],
        scratch_shapes=[pltpu.SMEM((1, 1), jnp.float32)],
    )(x, x.astype(jnp.bfloat16), embed, embed.astype(jnp.bfloat16))


@functools.cache
def _make_sc_gather():
    @functools.partial(
        pl.kernel,
        mesh=plsc.VectorSubcoreMesh(core_axis_name="c", subcore_axis_name="s"),
        compiler_params=pltpu.CompilerParams(use_tc_tiling_on_sc=False),
        out_type=jax.ShapeDtypeStruct((_B, _DIM), jnp.float32),
        scratch_types=[
            pltpu.VMEM((_NCH, _CH), jnp.int32),
            pltpu.VMEM((_BPW, _DIM), jnp.float32),
            pltpu.SemaphoreType.DMA,
        ],
    )
    def _sc_gather(table_hbm, idx_hbm, out_hbm, idx_v, rows_v, sem):
        wid = lax.axis_index("s") * 2 + lax.axis_index("c")
        base = wid * _BPW
        pltpu.sync_copy(idx_hbm.at[wid], idx_v)
        copies = []
        for j in range(_NCH):
            copies.append(pltpu.async_copy(
                table_hbm.at[idx_v.at[j]],
                rows_v.at[pl.ds(j * _CH, _CH)], sem))
        for c in copies:
            c.wait()
        pltpu.sync_copy(rows_v, out_hbm.at[pl.ds(base, _BPW)])

    return _sc_gather


def kernel(x, embed):
    ind2, dsum = _tc_argmin(x, embed)
    table = embed.T
    idx3 = ind2.reshape(_NW, _NCH, _CH)
    quantize = _make_sc_gather()(table, idx3)
    diff = dsum.reshape(())
    embed_ind = ind2.reshape(_B)
    return quantize, diff, embed_ind
